# baseline (speedup 1.0000x reference)
import functools
import jax
import jax.numpy as jnp
from jax import lax
from jax.experimental import pallas as pl
from jax.experimental.pallas import tpu as pltpu
from jax.experimental.pallas import tpu_sc as plsc

K_TOP_ = 32
L_ = 16
N_ = 32768
ROWS_ = 128
N_CORES_ = 2
N_SUBCORES_ = 16
N_WORKERS_ = N_CORES_ * N_SUBCORES_
RPW_ = ROWS_ // N_WORKERS_


_GATHER_DNUMS_ = lax.GatherDimensionNumbers(
    offset_dims=(), collapsed_slice_dims=(0,), start_index_map=(0,))


def _lane_perm(v, idx):
    return lax.gather(v, idx[:, None], _GATHER_DNUMS_, slice_sizes=(1,),
                      mode=lax.GatherScatterMode.PROMISE_IN_BOUNDS,
                      unique_indices=True)


def _vmax_splat(v, iota):
    for s in (1, 2, 4, 8):
        v = jnp.maximum(v, _lane_perm(v, iota ^ s))
    return v


def _sc_topk_body(x_hbm, out_hbm, rowbuf, candbuf, outbuf):
    wid = lax.axis_index("s") * N_CORES_ + lax.axis_index("c")
    iota = lax.iota(jnp.int32, L_)
    neg = jnp.full((L_,), -jnp.inf, jnp.float32)

    for r in range(RPW_):
        row = wid * RPW_ + r
        pltpu.sync_copy(x_hbm.at[row], rowbuf)

        def p1(i, accs):
            a0, a1, a2, a3 = accs
            b = i * (4 * L_)
            a0 = jnp.maximum(a0, rowbuf[pl.ds(b, L_)])
            a1 = jnp.maximum(a1, rowbuf[pl.ds(b + L_, L_)])
            a2 = jnp.maximum(a2, rowbuf[pl.ds(b + 2 * L_, L_)])
            a3 = jnp.maximum(a3, rowbuf[pl.ds(b + 3 * L_, L_)])
            return a0, a1, a2, a3

        a0, a1, a2, a3 = lax.fori_loop(0, N_ // (4 * L_), p1,
                                       (neg, neg, neg, neg))
        m01 = jnp.maximum(a0, a1)
        m23 = jnp.maximum(a2, a3)
        t0v = -_vmax_splat(-jnp.minimum(m01, m23), iota)

        def p2(i, ptr):
            v = rowbuf[pl.ds(i * L_, L_)]
            msk = v >= t0v

            def hit(p):
                plsc.store_compressed(candbuf.at[pl.ds(p, L_)], v, mask=msk)
                return p + jnp.sum(msk.astype(jnp.int32))

            return lax.cond(jnp.any(msk), hit, lambda p: p, ptr)

        ptr = lax.fori_loop(0, N_ // L_, p2, jnp.int32(0))
        candbuf[pl.ds(ptr, L_)] = neg
        ncv = ptr // L_ + 1

        def pa(j, acc):
            return jnp.maximum(acc, candbuf[pl.ds(j * L_, L_)])

        def emit_cond(i):
            return i < K_TOP_

        def emit(i):
            acc = lax.fori_loop(0, ncv, pa, neg)
            mv = _vmax_splat(acc, iota)

            def pb(j, cnt):
                v = candbuf[pl.ds(j * L_, L_)]
                eq = v == mv
                candbuf[pl.ds(j * L_, L_)] = jnp.where(eq, neg, v)
                return cnt + jnp.sum(eq.astype(jnp.int32))

            cnt = lax.fori_loop(0, ncv, pb, jnp.int32(0))
            pos0 = iota + i
            plsc.store_scatter(outbuf, [pos0], mv,
                               mask=(iota < cnt) & (pos0 < K_TOP_))
            pos1 = pos0 + L_
            plsc.store_scatter(outbuf, [pos1], mv,
                               mask=((iota + L_) < cnt) & (pos1 < K_TOP_))
            return i + cnt

        lax.while_loop(emit_cond, emit, jnp.int32(0))
        pltpu.sync_copy(outbuf, out_hbm.at[row])


@functools.lru_cache(maxsize=1)
def _build_sc_topk():
    return pl.kernel(
        _sc_topk_body,
        out_type=jax.ShapeDtypeStruct((ROWS_, K_TOP_), jnp.float32),
        mesh=plsc.VectorSubcoreMesh(core_axis_name="c", subcore_axis_name="s",
                                    num_cores=N_CORES_,
                                    num_subcores=N_SUBCORES_),
        scratch_types=[
            pltpu.VMEM((N_,), jnp.float32),
            pltpu.VMEM((N_ + L_,), jnp.float32),
            pltpu.VMEM((K_TOP_,), jnp.float32),
        ],
        compiler_params=pltpu.CompilerParams(needs_layout_passes=False),
    )


def kernel(inputs):
    return _build_sc_topk()(inputs)


# speedup vs baseline: 2.7864x; 2.7864x over previous
import functools
import jax
import jax.numpy as jnp
from jax import lax
from jax.experimental import pallas as pl
from jax.experimental.pallas import tpu as pltpu
from jax.experimental.pallas import tpu_sc as plsc

K_TOP_ = 32
L_ = 16
N_ = 32768
ROWS_ = 128
N_CORES_ = 2
N_SUBCORES_ = 16
N_WORKERS_ = N_CORES_ * N_SUBCORES_
RPW_ = ROWS_ // N_WORKERS_


_GATHER_DNUMS_ = lax.GatherDimensionNumbers(
    offset_dims=(), collapsed_slice_dims=(0,), start_index_map=(0,))


def _lane_perm(v, idx):
    return lax.gather(v, idx[:, None], _GATHER_DNUMS_, slice_sizes=(1,),
                      mode=lax.GatherScatterMode.PROMISE_IN_BOUNDS,
                      unique_indices=True)


def _vmax_splat(v, iota):
    for s in (1, 2, 4, 8):
        v = jnp.maximum(v, _lane_perm(v, iota ^ s))
    return v


def _sc_topk_body(x_hbm, out_hbm, rowbuf, candbuf, outbuf):
    wid = lax.axis_index("s") * N_CORES_ + lax.axis_index("c")
    iota = lax.iota(jnp.int32, L_)
    neg = jnp.full((L_,), -jnp.inf, jnp.float32)

    for r in range(RPW_):
        row = wid * RPW_ + r
        pltpu.sync_copy(x_hbm.at[row], rowbuf)

        P1U = 16

        def p1(i, accs):
            a0, a1, a2, a3 = accs
            b = i * (P1U * L_)
            for j in range(0, P1U, 4):
                a0 = jnp.maximum(a0, rowbuf[pl.ds(b + j * L_, L_)])
                a1 = jnp.maximum(a1, rowbuf[pl.ds(b + (j + 1) * L_, L_)])
                a2 = jnp.maximum(a2, rowbuf[pl.ds(b + (j + 2) * L_, L_)])
                a3 = jnp.maximum(a3, rowbuf[pl.ds(b + (j + 3) * L_, L_)])
            return a0, a1, a2, a3

        a0, a1, a2, a3 = lax.fori_loop(0, N_ // (P1U * L_), p1,
                                       (neg, neg, neg, neg))
        m01 = jnp.maximum(a0, a1)
        m23 = jnp.maximum(a2, a3)
        t0v = -_vmax_splat(-jnp.minimum(m01, m23), iota)

        P2U = 8

        def p2(i, ptr):
            b = i * (P2U * L_)
            vs = [rowbuf[pl.ds(b + j * L_, L_)] for j in range(P2U)]
            ms = [v >= t0v for v in vs]
            cs = [jnp.sum(m.astype(jnp.int32)) for m in ms]
            for j in range(P2U):
                plsc.store_compressed(candbuf.at[pl.ds(ptr, L_)], vs[j],
                                      mask=ms[j])
                ptr = ptr + cs[j]
            return ptr

        ptr = lax.fori_loop(0, N_ // (P2U * L_), p2, jnp.int32(0))
        candbuf[pl.ds(ptr, L_)] = neg
        ncv = ptr // L_ + 1

        def pa(j, acc):
            return jnp.maximum(acc, candbuf[pl.ds(j * L_, L_)])

        def emit_cond(i):
            return i < K_TOP_

        def emit(i):
            acc = lax.fori_loop(0, ncv, pa, neg)
            mv = _vmax_splat(acc, iota)

            def pb(j, cnt):
                v = candbuf[pl.ds(j * L_, L_)]
                eq = v == mv
                candbuf[pl.ds(j * L_, L_)] = jnp.where(eq, neg, v)
                return cnt + jnp.sum(eq.astype(jnp.int32))

            cnt = lax.fori_loop(0, ncv, pb, jnp.int32(0))
            pos0 = iota + i
            plsc.store_scatter(outbuf, [pos0], mv,
                               mask=(iota < cnt) & (pos0 < K_TOP_))
            pos1 = pos0 + L_
            plsc.store_scatter(outbuf, [pos1], mv,
                               mask=((iota + L_) < cnt) & (pos1 < K_TOP_))
            return i + cnt

        lax.while_loop(emit_cond, emit, jnp.int32(0))
        pltpu.sync_copy(outbuf, out_hbm.at[row])


@functools.lru_cache(maxsize=1)
def _build_sc_topk():
    return pl.kernel(
        _sc_topk_body,
        out_type=jax.ShapeDtypeStruct((ROWS_, K_TOP_), jnp.float32),
        mesh=plsc.VectorSubcoreMesh(core_axis_name="c", subcore_axis_name="s",
                                    num_cores=N_CORES_,
                                    num_subcores=N_SUBCORES_),
        scratch_types=[
            pltpu.VMEM((N_,), jnp.float32),
            pltpu.VMEM((N_ + L_,), jnp.float32),
            pltpu.VMEM((K_TOP_,), jnp.float32),
        ],
        compiler_params=pltpu.CompilerParams(needs_layout_passes=False),
    )


def kernel(inputs):
    return _build_sc_topk()(inputs)
